# Initial kernel scaffold; baseline (speedup 1.0000x reference)
import jax
import jax.numpy as jnp
from jax.experimental import pallas as pl


def kernel(emb, W1, b1, W2, b2, W3, b3, gamma, beta, edge_index, uids, iids, nids):
    raise NotImplementedError("write your pallas kernel here")


# speedup vs baseline: 8.7861x; 8.7861x over previous
import jax
import jax.numpy as jnp
from jax import lax
from jax.experimental import pallas as pl
from jax.experimental.pallas import tpu as pltpu
from jax.experimental.pallas import tpu_sc as plsc

_N = 10000
_D = 128
_E = 320000
_B = 16384
_NC = 2
_NS = 16
_NW = _NC * _NS
_EW = _E // _NW
_CH = 80
_NCH = _EW // _CH
_NRC = _N // _CH
_BW = _B // _NW

_mesh = plsc.VectorSubcoreMesh(
    core_axis_name="c", subcore_axis_name="s", num_cores=_NC, num_subcores=_NS
)


def _sc_deg_body(dst_ref, out_ref, acc, dst_c, ones_v, zero_v):
    c = lax.axis_index("c")
    s = lax.axis_index("s")
    wid = s * _NC + c
    for k in range(_CH // 16):
        ones_v[pl.ds(16 * k, 16)] = jnp.ones((16,), jnp.float32)
        zero_v[pl.ds(16 * k, 16)] = jnp.zeros((16,), jnp.float32)
    for k in range(8):
        ci = s + 16 * k

        @pl.when(ci < _NRC)
        def _():
            pltpu.sync_copy(zero_v, acc.at[pl.ds(pl.multiple_of(ci * _CH, 8), _CH)])

    base = pl.multiple_of(wid * _EW, 8)
    plsc.subcore_barrier()

    def body(j, carry):
        off = pl.multiple_of(base + j * _CH, 8)
        pltpu.sync_copy(dst_ref.at[pl.ds(off, _CH)], dst_c)
        pltpu.sync_copy(ones_v, acc.at[dst_c], add=True)
        return carry

    lax.fori_loop(0, _NCH, body, 0)
    plsc.subcore_barrier()
    obase = pl.multiple_of(c * _N, 8)
    for k in range(8):
        ci = s + 16 * k

        @pl.when(ci < _NRC)
        def _():
            off = pl.multiple_of(ci * _CH, 8)
            pltpu.sync_copy(acc.at[pl.ds(off, _CH)], zero_v)
            pltpu.sync_copy(zero_v, out_ref.at[pl.ds(obase + off, _CH)])


_sc_deg = pl.kernel(
    _sc_deg_body,
    out_type=jax.ShapeDtypeStruct((_NC * _N,), jnp.float32),
    mesh=_mesh,
    scratch_types=[
        pltpu.VMEM_SHARED((_N,), jnp.float32),
        pltpu.VMEM((_CH,), jnp.int32),
        pltpu.VMEM((_CH,), jnp.float32),
        pltpu.VMEM((_CH,), jnp.float32),
    ],
)


def _sc_agg_body(h_ref, src_ref, dst_ref, out_ref, acc, src_c, dst_c, rows,
                 zrows, sem):
    c = lax.axis_index("c")
    s = lax.axis_index("s")
    wid = s * _NC + c

    def zb(i, carry):
        for k in range(_D // 16):
            zrows[i, pl.ds(16 * k, 16)] = jnp.zeros((16,), jnp.float32)
        return carry

    lax.fori_loop(0, _CH, zb, 0)
    for k in range(8):
        ci = s + 16 * k

        @pl.when(ci < _NRC)
        def _():
            pltpu.sync_copy(zrows, acc.at[pl.ds(pl.multiple_of(ci * _CH, 8), _CH), :])

    base = pl.multiple_of(wid * _EW, 8)
    plsc.subcore_barrier()

    def body(j, carry):
        off = pl.multiple_of(base + j * _CH, 8)
        pltpu.sync_copy(src_ref.at[pl.ds(off, _CH)], src_c)
        pltpu.sync_copy(dst_ref.at[pl.ds(off, _CH)], dst_c)
        pltpu.async_copy(h_ref.at[src_c], rows, sem).wait()
        pltpu.sync_copy(rows, acc.at[dst_c], add=True)
        return carry

    lax.fori_loop(0, _NCH, body, 0)
    plsc.subcore_barrier()
    for k in range(8):
        ci = s + 16 * k

        @pl.when(ci < _NRC)
        def _():
            off = pl.multiple_of(ci * _CH, 8)
            pltpu.sync_copy(acc.at[pl.ds(off, _CH), :], zrows)
            pltpu.sync_copy(zrows, out_ref.at[c, pl.ds(off, _CH), :])


_sc_agg = pl.kernel(
    _sc_agg_body,
    out_type=jax.ShapeDtypeStruct((_NC, _N, _D), jnp.float32),
    mesh=_mesh,
    scratch_types=[
        pltpu.VMEM_SHARED((_N, _D), jnp.float32),
        pltpu.VMEM((_CH,), jnp.int32),
        pltpu.VMEM((_CH,), jnp.int32),
        pltpu.VMEM((_CH, _D), jnp.float32),
        pltpu.VMEM((_CH, _D), jnp.float32),
        pltpu.SemaphoreType.DMA,
    ],
)


def _sc_gth_body(x_ref, u_ref, i_ref, n_ref, out_ref, idx_c, rows, sem):
    c = lax.axis_index("c")
    s = lax.axis_index("s")
    wid = s * _NC + c
    for t, ids_ref in enumerate((u_ref, i_ref, n_ref)):
        for j in range(_BW // 128):
            off = pl.multiple_of(wid * _BW + j * 128, 8)
            pltpu.sync_copy(ids_ref.at[pl.ds(off, 128)], idx_c)
            pltpu.async_copy(x_ref.at[idx_c], rows, sem).wait()
            pltpu.sync_copy(rows, out_ref.at[t, pl.ds(off, 128), :])


_sc_gth = pl.kernel(
    _sc_gth_body,
    out_type=jax.ShapeDtypeStruct((3, _B, _D), jnp.float32),
    mesh=_mesh,
    scratch_types=[
        pltpu.VMEM((128,), jnp.int32),
        pltpu.VMEM((128, _D), jnp.float32),
        pltpu.SemaphoreType.DMA,
    ],
)


def _tc_prep_body(deg0_ref, deg1_ref, emb_ref, w_ref, h_ref, dinv_ref):
    deg = deg0_ref[...] + deg1_ref[...]
    dinv = lax.rsqrt(jnp.maximum(deg, 1.0))
    dinv_ref[...] = dinv
    h = jnp.dot(emb_ref[...], w_ref[...], preferred_element_type=jnp.float32)
    h_ref[...] = h * dinv[:, None]


_tc_prep = pl.pallas_call(
    _tc_prep_body,
    out_shape=(
        jax.ShapeDtypeStruct((_N, _D), jnp.float32),
        jax.ShapeDtypeStruct((_N,), jnp.float32),
    ),
)


def _tc_mid_body(part_ref, dinv_ref, b_ref, g_ref, bt_ref, w_ref, out_ref):
    dinv = dinv_ref[...]
    x = (part_ref[0] + part_ref[1]) * dinv[:, None] + b_ref[...][None, :]
    mean = jnp.mean(x, axis=0)
    var = jnp.mean(jnp.square(x - mean[None, :]), axis=0)
    y = (x - mean[None, :]) * lax.rsqrt(var + 1e-5)[None, :]
    y = y * g_ref[...][None, :] + bt_ref[...][None, :]
    e = jnp.where(y > 0.0, y, jnp.exp(y) - 1.0)
    out_ref[...] = jnp.dot(
        e, w_ref[...], preferred_element_type=jnp.float32) * dinv[:, None]


_tc_mid = pl.pallas_call(
    _tc_mid_body,
    out_shape=jax.ShapeDtypeStruct((_N, _D), jnp.float32),
)


def _tc_fin_body(part_ref, dinv_ref, b_ref, out_ref):
    out_ref[...] = ((part_ref[0] + part_ref[1]) * dinv_ref[...][:, None]
                    + b_ref[...][None, :])


_tc_fin = pl.pallas_call(
    _tc_fin_body,
    out_shape=jax.ShapeDtypeStruct((_N, _D), jnp.float32),
)


def _tc_score_body(g_ref, sp_ref, sn_ref):
    gu = g_ref[0]
    sp_ref[...] = jnp.sum(gu * g_ref[1], axis=1)
    sn_ref[...] = jnp.sum(gu * g_ref[2], axis=1)


_tc_score = pl.pallas_call(
    _tc_score_body,
    out_shape=(
        jax.ShapeDtypeStruct((_B,), jnp.float32),
        jax.ShapeDtypeStruct((_B,), jnp.float32),
    ),
)


def kernel(emb, W1, b1, W2, b2, W3, b3, gamma, beta, edge_index, uids, iids, nids):
    edge_index = edge_index.astype(jnp.int32)
    src = edge_index[0]
    dst = edge_index[1]
    uids = uids.astype(jnp.int32)
    iids = iids.astype(jnp.int32)
    nids = nids.astype(jnp.int32)

    degp = _sc_deg(dst)
    h1, dinv = _tc_prep(degp[:_N], degp[_N:], emb, W1)
    p1 = _sc_agg(h1, src, dst)
    h2 = _tc_mid(p1, dinv, b1, gamma, beta, W2)
    p2 = _sc_agg(h2, src, dst)
    h3 = _tc_mid(p2, dinv, b2, gamma, beta, W3)
    p3 = _sc_agg(h3, src, dst)
    x3 = _tc_fin(p3, dinv, b3)
    g = _sc_gth(x3, uids, iids, nids)
    sp, sn = _tc_score(g)
    return sp[:, None], sn[:, None]


# speedup vs baseline: 16.3750x; 1.8637x over previous
import jax
import jax.numpy as jnp
from jax import lax
from jax.experimental import pallas as pl
from jax.experimental.pallas import tpu as pltpu
from jax.experimental.pallas import tpu_sc as plsc

_N = 10000
_D = 128
_E = 320000
_B = 16384
_NC = 2
_NS = 16
_NW = _NC * _NS
_EW = _E // _NW
_CH = 80
_NCH = _EW // _CH
_NRC = _N // _CH
_BW = _B // _NW

_mesh = plsc.VectorSubcoreMesh(
    core_axis_name="c", subcore_axis_name="s", num_cores=_NC, num_subcores=_NS
)


def _sc_deg_body(dst_ref, out_ref, acc, dst_c, ones_v, zero_v, isem, ssem):
    c = lax.axis_index("c")
    s = lax.axis_index("s")
    wid = s * _NC + c
    for k in range(_CH // 16):
        ones_v[pl.ds(16 * k, 16)] = jnp.ones((16,), jnp.float32)
        zero_v[pl.ds(16 * k, 16)] = jnp.zeros((16,), jnp.float32)
    for k in range(8):
        ci = s + 16 * k

        @pl.when(ci < _NRC)
        def _():
            pltpu.sync_copy(zero_v, acc.at[pl.ds(pl.multiple_of(ci * _CH, 8), _CH)])

    base = pl.multiple_of(wid * _EW, 8)

    def ld(j, carry):
        off = pl.multiple_of(base + j * _CH, 8)
        pltpu.async_copy(dst_ref.at[pl.ds(off, _CH)], dst_c.at[j], isem)
        return carry

    lax.fori_loop(0, _NCH, ld, 0)

    def ldw(j, carry):
        pltpu.make_async_copy(dst_ref.at[pl.ds(base, _CH)], dst_c.at[0],
                              isem).wait()
        return carry

    lax.fori_loop(0, _NCH, ldw, 0)
    plsc.subcore_barrier()

    def body(j, carry):
        pltpu.async_copy(ones_v, acc.at[dst_c.at[j]], ssem, add=True)
        return carry

    lax.fori_loop(0, _NCH, body, 0)

    def bodyw(j, carry):
        pltpu.make_async_copy(ones_v, acc.at[dst_c.at[0]], ssem).wait()
        return carry

    lax.fori_loop(0, _NCH, bodyw, 0)
    plsc.subcore_barrier()
    obase = pl.multiple_of(c * _N, 8)
    for k in range(8):
        ci = s + 16 * k

        @pl.when(ci < _NRC)
        def _():
            off = pl.multiple_of(ci * _CH, 8)
            pltpu.sync_copy(acc.at[pl.ds(off, _CH)], zero_v)
            pltpu.sync_copy(zero_v, out_ref.at[pl.ds(obase + off, _CH)])


_sc_deg = pl.kernel(
    _sc_deg_body,
    out_type=jax.ShapeDtypeStruct((_NC * _N,), jnp.float32),
    mesh=_mesh,
    scratch_types=[
        pltpu.VMEM_SHARED((_N,), jnp.float32),
        pltpu.VMEM((_NCH, _CH), jnp.int32),
        pltpu.VMEM((_CH,), jnp.float32),
        pltpu.VMEM((_CH,), jnp.float32),
        pltpu.SemaphoreType.DMA,
        pltpu.SemaphoreType.DMA,
    ],
)


def _sc_agg_body(h_ref, src_ref, dst_ref, out_ref, acc, src_v, dst_c, rows,
                 zrows, gsem, ssem, isem):
    c = lax.axis_index("c")
    s = lax.axis_index("s")
    wid = s * _NC + c
    base = pl.multiple_of(wid * _EW, 8)

    def ld(j, carry):
        off = pl.multiple_of(base + j * _CH, 8)
        pltpu.async_copy(dst_ref.at[pl.ds(off, _CH)], dst_c.at[j], isem)
        return carry

    lax.fori_loop(0, _NCH, ld, 0)
    pltpu.async_copy(src_ref.at[pl.ds(base, _EW)], src_v, isem)

    def zb(i, carry):
        for k in range(_D // 16):
            zrows[i, pl.ds(16 * k, 16)] = jnp.zeros((16,), jnp.float32)
        return carry

    lax.fori_loop(0, 16, zb, 0)

    def zc(k, carry):
        ci = s + 16 * k
        pltpu.sync_copy(zrows, acc.at[pl.ds(pl.multiple_of(ci * 16, 8), 16), :])
        return carry

    lax.fori_loop(0, jnp.where(s < (_N // 16) % 16, _N // 256 + 1, _N // 256),
                  zc, 0)

    def ldw(j, carry):
        pltpu.make_async_copy(dst_ref.at[pl.ds(base, _CH)], dst_c.at[0],
                              isem).wait()
        return carry

    lax.fori_loop(0, _NCH, ldw, 0)
    pltpu.make_async_copy(src_ref.at[pl.ds(base, _EW)], src_v, isem).wait()
    plsc.subcore_barrier()

    def _gather_start(j, b):
        pltpu.async_copy(h_ref.at[src_v.at[pl.ds(j * _CH, _CH)]], rows.at[b],
                         gsem)

    def _gather_wait(b):
        pltpu.make_async_copy(h_ref.at[src_v.at[pl.ds(0, _CH)]], rows.at[b],
                              gsem).wait()

    def _scatter_wait(b):
        pltpu.make_async_copy(rows.at[b], acc.at[dst_c.at[0]], ssem).wait()

    _gather_start(0, 0)

    def body(j, carry):
        b = lax.rem(j, 2)
        _gather_wait(b)

        @pl.when(j >= 1)
        def _():
            _scatter_wait(1 - b)

        @pl.when(j <= _NCH - 2)
        def _():
            _gather_start(j + 1, 1 - b)

        pltpu.async_copy(rows.at[b], acc.at[dst_c.at[j]], ssem, add=True)
        return carry

    lax.fori_loop(0, _NCH, body, 0)
    _scatter_wait((_NCH - 1) % 2)
    plsc.subcore_barrier()
    for k in range(8):
        ci = s + 16 * k

        @pl.when(ci < _NRC)
        def _():
            off = pl.multiple_of(ci * _CH, 8)
            pltpu.sync_copy(acc.at[pl.ds(off, _CH), :], rows.at[k % 2])
            pltpu.sync_copy(rows.at[k % 2], out_ref.at[c, pl.ds(off, _CH), :])


_sc_agg = pl.kernel(
    _sc_agg_body,
    out_type=jax.ShapeDtypeStruct((_NC, _N, _D), jnp.float32),
    mesh=_mesh,
    scratch_types=[
        pltpu.VMEM_SHARED((_N, _D), jnp.float32),
        pltpu.VMEM((_EW,), jnp.int32),
        pltpu.VMEM((_NCH, _CH), jnp.int32),
        pltpu.VMEM((2, _CH, _D), jnp.float32),
        pltpu.VMEM((16, _D), jnp.float32),
        pltpu.SemaphoreType.DMA,
        pltpu.SemaphoreType.DMA,
        pltpu.SemaphoreType.DMA,
    ],
)


def _sc_gth_body(x_ref, u_ref, i_ref, n_ref, out_ref, idx_c, rows, sem):
    c = lax.axis_index("c")
    s = lax.axis_index("s")
    wid = s * _NC + c
    for t, ids_ref in enumerate((u_ref, i_ref, n_ref)):
        for j in range(_BW // 128):
            off = pl.multiple_of(wid * _BW + j * 128, 8)
            pltpu.sync_copy(ids_ref.at[pl.ds(off, 128)], idx_c)
            pltpu.async_copy(x_ref.at[idx_c], rows, sem).wait()
            pltpu.sync_copy(rows, out_ref.at[t, pl.ds(off, 128), :])


_sc_gth = pl.kernel(
    _sc_gth_body,
    out_type=jax.ShapeDtypeStruct((3, _B, _D), jnp.float32),
    mesh=_mesh,
    scratch_types=[
        pltpu.VMEM((128,), jnp.int32),
        pltpu.VMEM((128, _D), jnp.float32),
        pltpu.SemaphoreType.DMA,
    ],
)


def _tc_prep_body(deg0_ref, deg1_ref, emb_ref, w_ref, h_ref, dinv_ref):
    deg = deg0_ref[...] + deg1_ref[...]
    dinv = lax.rsqrt(jnp.maximum(deg, 1.0))
    dinv_ref[...] = dinv
    h = jnp.dot(emb_ref[...], w_ref[...], preferred_element_type=jnp.float32)
    h_ref[...] = h * dinv[:, None]


_tc_prep = pl.pallas_call(
    _tc_prep_body,
    out_shape=(
        jax.ShapeDtypeStruct((_N, _D), jnp.float32),
        jax.ShapeDtypeStruct((_N,), jnp.float32),
    ),
)


def _tc_mid_body(part_ref, dinv_ref, b_ref, g_ref, bt_ref, w_ref, out_ref):
    dinv = dinv_ref[...]
    x = (part_ref[0] + part_ref[1]) * dinv[:, None] + b_ref[...][None, :]
    mean = jnp.mean(x, axis=0)
    var = jnp.mean(jnp.square(x - mean[None, :]), axis=0)
    y = (x - mean[None, :]) * lax.rsqrt(var + 1e-5)[None, :]
    y = y * g_ref[...][None, :] + bt_ref[...][None, :]
    e = jnp.where(y > 0.0, y, jnp.exp(y) - 1.0)
    out_ref[...] = jnp.dot(
        e, w_ref[...], preferred_element_type=jnp.float32) * dinv[:, None]


_tc_mid = pl.pallas_call(
    _tc_mid_body,
    out_shape=jax.ShapeDtypeStruct((_N, _D), jnp.float32),
)


def _tc_fin_body(part_ref, dinv_ref, b_ref, out_ref):
    out_ref[...] = ((part_ref[0] + part_ref[1]) * dinv_ref[...][:, None]
                    + b_ref[...][None, :])


_tc_fin = pl.pallas_call(
    _tc_fin_body,
    out_shape=jax.ShapeDtypeStruct((_N, _D), jnp.float32),
)


def _tc_score_body(g_ref, sp_ref, sn_ref):
    gu = g_ref[0]
    sp_ref[...] = jnp.sum(gu * g_ref[1], axis=1)
    sn_ref[...] = jnp.sum(gu * g_ref[2], axis=1)


_tc_score = pl.pallas_call(
    _tc_score_body,
    out_shape=(
        jax.ShapeDtypeStruct((_B,), jnp.float32),
        jax.ShapeDtypeStruct((_B,), jnp.float32),
    ),
)


def kernel(emb, W1, b1, W2, b2, W3, b3, gamma, beta, edge_index, uids, iids, nids):
    edge_index = edge_index.astype(jnp.int32)
    src = edge_index[0]
    dst = edge_index[1]
    uids = uids.astype(jnp.int32)
    iids = iids.astype(jnp.int32)
    nids = nids.astype(jnp.int32)

    degp = _sc_deg(dst)
    h1, dinv = _tc_prep(degp[:_N], degp[_N:], emb, W1)
    p1 = _sc_agg(h1, src, dst)
    h2 = _tc_mid(p1, dinv, b1, gamma, beta, W2)
    p2 = _sc_agg(h2, src, dst)
    h3 = _tc_mid(p2, dinv, b2, gamma, beta, W3)
    p3 = _sc_agg(h3, src, dst)
    x3 = _tc_fin(p3, dinv, b3)
    g = _sc_gth(x3, uids, iids, nids)
    sp, sn = _tc_score(g)
    return sp[:, None], sn[:, None]


# speedup vs baseline: 23.6053x; 1.4415x over previous
import jax
import jax.numpy as jnp
from jax import lax
from jax.experimental import pallas as pl
from jax.experimental.pallas import tpu as pltpu
from jax.experimental.pallas import tpu_sc as plsc

_N = 10000
_D = 128
_E = 320000
_B = 16384
_NC = 2
_NS = 16
_NW = _NC * _NS
_EW = _E // _NW
_CH = 80
_NCH = _EW // _CH
_NRC = _N // _CH
_BW = _B // _NW

_mesh = plsc.VectorSubcoreMesh(
    core_axis_name="c", subcore_axis_name="s", num_cores=_NC, num_subcores=_NS
)


def _sc_deg_body(dst_ref, out_ref, acc, dst_c, ones_v, zero_v, isem, ssem):
    c = lax.axis_index("c")
    s = lax.axis_index("s")
    wid = s * _NC + c
    for k in range(_CH // 16):
        ones_v[pl.ds(16 * k, 16)] = jnp.ones((16,), jnp.float32)
        zero_v[pl.ds(16 * k, 16)] = jnp.zeros((16,), jnp.float32)
    for k in range(8):
        ci = s + 16 * k

        @pl.when(ci < _NRC)
        def _():
            pltpu.sync_copy(zero_v, acc.at[pl.ds(pl.multiple_of(ci * _CH, 8), _CH)])

    base = pl.multiple_of(wid * _EW, 8)

    def ld(j, carry):
        off = pl.multiple_of(base + j * _CH, 8)
        pltpu.async_copy(dst_ref.at[pl.ds(off, _CH)], dst_c.at[j], isem)
        return carry

    lax.fori_loop(0, _NCH, ld, 0)

    def ldw(j, carry):
        pltpu.make_async_copy(dst_ref.at[pl.ds(base, _CH)], dst_c.at[0],
                              isem).wait()
        return carry

    lax.fori_loop(0, _NCH, ldw, 0)
    plsc.subcore_barrier()

    def body(j, carry):
        pltpu.async_copy(ones_v, acc.at[dst_c.at[j]], ssem, add=True)
        return carry

    lax.fori_loop(0, _NCH, body, 0)

    def bodyw(j, carry):
        pltpu.make_async_copy(ones_v, acc.at[dst_c.at[0]], ssem).wait()
        return carry

    lax.fori_loop(0, _NCH, bodyw, 0)
    plsc.subcore_barrier()
    obase = pl.multiple_of(c * _N, 8)
    for k in range(8):
        ci = s + 16 * k

        @pl.when(ci < _NRC)
        def _():
            off = pl.multiple_of(ci * _CH, 8)
            pltpu.sync_copy(acc.at[pl.ds(off, _CH)], zero_v)
            pltpu.sync_copy(zero_v, out_ref.at[pl.ds(obase + off, _CH)])


_sc_deg = pl.kernel(
    _sc_deg_body,
    out_type=jax.ShapeDtypeStruct((_NC * _N,), jnp.float32),
    mesh=_mesh,
    scratch_types=[
        pltpu.VMEM_SHARED((_N,), jnp.float32),
        pltpu.VMEM((_NCH, _CH), jnp.int32),
        pltpu.VMEM((_CH,), jnp.float32),
        pltpu.VMEM((_CH,), jnp.float32),
        pltpu.SemaphoreType.DMA,
        pltpu.SemaphoreType.DMA,
    ],
)


def _sc_agg_body(h_ref, src_ref, dst_ref, out_ref, acc, srcr, dstr, rows,
                 zrows, gsemA, gsemB, ssemA, ssemB, isem, dsem):
    c = lax.axis_index("c")
    s = lax.axis_index("s")
    wid = s * _NC + c
    base = pl.multiple_of(wid * _EW, 8)

    def zb(i, carry):
        for k in range(_D // 16):
            zrows[i, pl.ds(16 * k, 16)] = jnp.zeros((16,), jnp.float32)
        return carry

    lax.fori_loop(0, 16, zb, 0)

    def zc(k, carry):
        ci = s + 16 * k
        pltpu.sync_copy(zrows, acc.at[pl.ds(pl.multiple_of(ci * 16, 8), 16), :])
        return carry

    lax.fori_loop(0, jnp.where(s < (_N // 16) % 16, _N // 256 + 1, _N // 256),
                  zc, 0)

    def _idx_fire(j, m):
        off = pl.multiple_of(base + j * _CH, 8)
        pltpu.async_copy(src_ref.at[pl.ds(off, _CH)], srcr.at[m], isem)
        pltpu.async_copy(dst_ref.at[pl.ds(off, _CH)], dstr.at[lax.rem(j, 5)],
                         dsem)

    def _idx_wait(m):
        pltpu.make_async_copy(src_ref.at[pl.ds(base, _CH)], srcr.at[m],
                              isem).wait()
        pltpu.make_async_copy(dst_ref.at[pl.ds(base, _CH)], dstr.at[0],
                              dsem).wait()

    def _gather_start(m, sem):
        pltpu.async_copy(h_ref.at[srcr.at[m]], rows.at[m], sem)

    def _gather_wait(m, sem):
        pltpu.make_async_copy(h_ref.at[srcr.at[0]], rows.at[m], sem).wait()

    def _scatter_start(j, m, sem):
        pltpu.async_copy(rows.at[m], acc.at[dstr.at[lax.rem(j, 5)]], sem,
                         add=True)

    def _scatter_wait(sem):
        pltpu.make_async_copy(rows.at[0], acc.at[dstr.at[0]], sem).wait()

    _idx_fire(0, 0)
    _idx_fire(1, 1)
    _idx_wait(0)
    _idx_wait(1)
    _gather_start(0, gsemA)
    _gather_start(1, gsemB)
    _idx_fire(2, 2)
    plsc.subcore_barrier()

    def _step(j, gsem, ssem):
        m = lax.rem(j, 4)
        _gather_wait(m, gsem)

        @pl.when(j >= 2)
        def _():
            _scatter_wait(ssem)

        @pl.when(j <= _NCH - 3)
        def _():
            _idx_wait(lax.rem(j + 2, 4))
            _gather_start(lax.rem(j + 2, 4), gsem)

        @pl.when(j <= _NCH - 4)
        def _():
            _idx_fire(j + 3, lax.rem(j + 3, 4))

        _scatter_start(j, m, ssem)

    def body(i, carry):
        _step(2 * i, gsemA, ssemA)
        _step(2 * i + 1, gsemB, ssemB)
        return carry

    lax.fori_loop(0, (_NCH - 1) // 2, body, 0)
    _step(_NCH - 1, gsemA, ssemA)
    _scatter_wait(ssemB)
    _scatter_wait(ssemA)
    plsc.subcore_barrier()
    for k in range(8):
        ci = s + 16 * k

        @pl.when(ci < _NRC)
        def _():
            off = pl.multiple_of(ci * _CH, 8)
            buf = rows.at[k % 2].at[pl.ds(0, _CH), :]
            pltpu.sync_copy(acc.at[pl.ds(off, _CH), :], buf)
            pltpu.sync_copy(buf, out_ref.at[c, pl.ds(off, _CH), :])


_sc_agg = pl.kernel(
    _sc_agg_body,
    out_type=jax.ShapeDtypeStruct((_NC, _N, _D), jnp.float32),
    mesh=_mesh,
    scratch_types=[
        pltpu.VMEM_SHARED((_N, _D), jnp.float32),
        pltpu.VMEM((4, _CH), jnp.int32),
        pltpu.VMEM((5, _CH), jnp.int32),
        pltpu.VMEM((4, _CH, _D), jnp.float32),
        pltpu.VMEM((16, _D), jnp.float32),
        pltpu.SemaphoreType.DMA,
        pltpu.SemaphoreType.DMA,
        pltpu.SemaphoreType.DMA,
        pltpu.SemaphoreType.DMA,
        pltpu.SemaphoreType.DMA,
        pltpu.SemaphoreType.DMA,
    ],
)


def _sc_gth_body(x_ref, u_ref, i_ref, n_ref, out_ref, idx_u, idx_i, idx_n,
                 rows, gsemA, gsemB, wsemA, wsemB, isem):
    c = lax.axis_index("c")
    s = lax.axis_index("s")
    wid = s * _NC + c
    base = pl.multiple_of(wid * _BW, 8)
    refs = (u_ref, i_ref, n_ref)
    bufs = (idx_u, idx_i, idx_n)
    for t in range(3):
        pltpu.async_copy(refs[t].at[pl.ds(base, _BW)], bufs[t], isem)
    for t in range(3):
        pltpu.make_async_copy(refs[t].at[pl.ds(base, _BW)], bufs[t],
                              isem).wait()

    seq = [(t, j) for t in range(3) for j in range(_BW // 128)]
    gsems = (gsemA, gsemB)
    wsems = (wsemA, wsemB)

    def gstart(k):
        t, j = seq[k]
        pltpu.async_copy(x_ref.at[bufs[t].at[pl.ds(j * 128, 128)]],
                         rows.at[k % 4], gsems[k % 2])

    def gwait(k):
        pltpu.make_async_copy(x_ref.at[idx_u.at[pl.ds(0, 128)]],
                              rows.at[k % 4], gsems[k % 2]).wait()

    def wstart(k):
        t, j = seq[k]
        pltpu.async_copy(rows.at[k % 4],
                         out_ref.at[t, pl.ds(wid * _BW + j * 128, 128), :],
                         wsems[k % 2])

    def wwait(k):
        pltpu.make_async_copy(rows.at[k % 4],
                              out_ref.at[0, pl.ds(0, 128), :],
                              wsems[k % 2]).wait()

    gstart(0)
    gstart(1)
    for k in range(len(seq)):
        gwait(k)
        if k >= 2:
            wwait(k - 2)
        if k + 2 < len(seq):
            gstart(k + 2)
        wstart(k)
    wwait(len(seq) - 2)
    wwait(len(seq) - 1)


_sc_gth = pl.kernel(
    _sc_gth_body,
    out_type=jax.ShapeDtypeStruct((3, _B, _D), jnp.float32),
    mesh=_mesh,
    scratch_types=[
        pltpu.VMEM((_BW,), jnp.int32),
        pltpu.VMEM((_BW,), jnp.int32),
        pltpu.VMEM((_BW,), jnp.int32),
        pltpu.VMEM((4, 128, _D), jnp.float32),
        pltpu.SemaphoreType.DMA,
        pltpu.SemaphoreType.DMA,
        pltpu.SemaphoreType.DMA,
        pltpu.SemaphoreType.DMA,
        pltpu.SemaphoreType.DMA,
    ],
)


def _tc_prep_body(deg0_ref, deg1_ref, emb_ref, w_ref, h_ref, dinv_ref):
    deg = deg0_ref[...] + deg1_ref[...]
    dinv = lax.rsqrt(jnp.maximum(deg, 1.0))
    dinv_ref[...] = dinv
    h = jnp.dot(emb_ref[...], w_ref[...], preferred_element_type=jnp.float32)
    h_ref[...] = h * dinv[:, None]


_tc_prep = pl.pallas_call(
    _tc_prep_body,
    out_shape=(
        jax.ShapeDtypeStruct((_N, _D), jnp.float32),
        jax.ShapeDtypeStruct((_N,), jnp.float32),
    ),
)


def _tc_mid_body(part_ref, dinv_ref, b_ref, g_ref, bt_ref, w_ref, out_ref):
    dinv = dinv_ref[...]
    x = (part_ref[0] + part_ref[1]) * dinv[:, None] + b_ref[...][None, :]
    mean = jnp.mean(x, axis=0)
    var = jnp.mean(jnp.square(x - mean[None, :]), axis=0)
    y = (x - mean[None, :]) * lax.rsqrt(var + 1e-5)[None, :]
    y = y * g_ref[...][None, :] + bt_ref[...][None, :]
    e = jnp.where(y > 0.0, y, jnp.exp(y) - 1.0)
    out_ref[...] = jnp.dot(
        e, w_ref[...], preferred_element_type=jnp.float32) * dinv[:, None]


_tc_mid = pl.pallas_call(
    _tc_mid_body,
    out_shape=jax.ShapeDtypeStruct((_N, _D), jnp.float32),
)


def _tc_fin_body(part_ref, dinv_ref, b_ref, out_ref):
    out_ref[...] = ((part_ref[0] + part_ref[1]) * dinv_ref[...][:, None]
                    + b_ref[...][None, :])


_tc_fin = pl.pallas_call(
    _tc_fin_body,
    out_shape=jax.ShapeDtypeStruct((_N, _D), jnp.float32),
)


def _tc_score_body(g_ref, sp_ref, sn_ref):
    gu = g_ref[0]
    sp_ref[...] = jnp.sum(gu * g_ref[1], axis=1)
    sn_ref[...] = jnp.sum(gu * g_ref[2], axis=1)


_tc_score = pl.pallas_call(
    _tc_score_body,
    out_shape=(
        jax.ShapeDtypeStruct((_B,), jnp.float32),
        jax.ShapeDtypeStruct((_B,), jnp.float32),
    ),
)


def kernel(emb, W1, b1, W2, b2, W3, b3, gamma, beta, edge_index, uids, iids, nids):
    edge_index = edge_index.astype(jnp.int32)
    src = edge_index[0]
    dst = edge_index[1]
    uids = uids.astype(jnp.int32)
    iids = iids.astype(jnp.int32)
    nids = nids.astype(jnp.int32)

    degp = _sc_deg(dst)
    h1, dinv = _tc_prep(degp[:_N], degp[_N:], emb, W1)
    p1 = _sc_agg(h1, src, dst)
    h2 = _tc_mid(p1, dinv, b1, gamma, beta, W2)
    p2 = _sc_agg(h2, src, dst)
    h3 = _tc_mid(p2, dinv, b2, gamma, beta, W3)
    p3 = _sc_agg(h3, src, dst)
    x3 = _tc_fin(p3, dinv, b3)
    g = _sc_gth(x3, uids, iids, nids)
    sp, sn = _tc_score(g)
    return sp[:, None], sn[:, None]
